# baseline (speedup 1.0000x reference)
import jax
import jax.numpy as jnp
from jax import lax
from jax.experimental import pallas as pl
from jax.experimental.pallas import tpu as pltpu
from jax.experimental.pallas import tpu_sc as plsc

_EMB = 64
_SCALE = 8.0
_W = 128
_NBUF = 4
_NC, _NS = 2, 16
_NWORK = _NC * _NS
_LANES = 16


def kernel(x, table):
    B, S = x.shape
    V, E = table.shape
    CT = B // _W
    n_wins = S * CT
    n_win = n_wins // _NWORK
    tab2 = jnp.pad(table, ((0, 0), (0, 2 * E - E)))
    xt = jnp.transpose(x).reshape(n_wins, _W)
    mesh = plsc.VectorSubcoreMesh(core_axis_name="c", subcore_axis_name="s")

    @pl.kernel(
        out_type=jax.ShapeDtypeStruct((S, E // 8, CT, 8, _W), table.dtype),
        mesh=mesh,
        compiler_params=pltpu.CompilerParams(
            use_tc_tiling_on_sc=False, needs_layout_passes=False
        ),
        scratch_types=(
            [pltpu.VMEM((n_win, _W), jnp.int32)]
            + [pltpu.VMEM((_W,), jnp.int32) for _ in range(_NBUF)]
            + [pltpu.VMEM((_W, 2 * E), jnp.float32) for _ in range(_NBUF)]
            + [pltpu.VMEM((E // 8, 8, _W), jnp.float32) for _ in range(_NBUF)]
            + [pltpu.SemaphoreType.DMA for _ in range(2 * _NBUF + 1)]
        ),
    )
    def _gather(tab_hbm, i_hbm, o_hbm, idx_v, *scr):
        hbuf = scr[:_NBUF]
        gbuf = scr[_NBUF : 2 * _NBUF]
        obuf = scr[2 * _NBUF : 3 * _NBUF]
        gsem = scr[3 * _NBUF : 4 * _NBUF]
        osem = scr[4 * _NBUF : 5 * _NBUF]
        isem = scr[5 * _NBUF]

        wid = lax.axis_index("s") * _NC + lax.axis_index("c")
        win0 = wid * n_win

        pltpu.async_copy(i_hbm.at[pl.ds(win0, n_win)], idx_v, isem).wait()

        iota = lax.iota(jnp.int32, _LANES)
        rowidx = [iota + k * _LANES for k in range(_W // _LANES)]

        def fill_hbuf(b, g):
            for k in range(_W // _LANES):
                sl = pl.ds(k * _LANES, _LANES)
                hbuf[b].at[sl][...] = idx_v.at[g, sl][...]

        def start_gather(b):
            pltpu.make_async_copy(tab_hbm.at[hbuf[b]], gbuf[b], gsem[b]).start()

        def out_dst(g):
            w = win0 + g
            s = w // CT
            ct = lax.rem(w, CT)
            return o_hbm.at[s, :, ct]

        for b in range(_NBUF):
            fill_hbuf(b, b)
            start_gather(b)

        @pl.loop(0, n_win, step=_NBUF)
        def _round(t):
            for b in range(_NBUF):
                g = t + b
                pltpu.make_async_copy(
                    tab_hbm.at[hbuf[b]], gbuf[b], gsem[b]
                ).wait()

                @pl.when(t > 0)
                def _():
                    pltpu.make_async_copy(obuf[b], out_dst(g), osem[b]).wait()

                @pl.loop(0, E // 8)
                def _ftile(rt):
                    f0 = rt * 8
                    for sub in range(8):
                        fvec = lax.broadcast(f0 + sub, (_LANES,))
                        for k in range(_W // _LANES):
                            v = plsc.load_gather(gbuf[b], [rowidx[k], fvec])
                            obuf[b].at[
                                rt, sub, pl.ds(k * _LANES, _LANES)
                            ][...] = (v * _SCALE)

                @pl.when(g + _NBUF < n_win)
                def _():
                    fill_hbuf(b, g + _NBUF)
                    start_gather(b)

                pltpu.make_async_copy(obuf[b], out_dst(g), osem[b]).start()

        for b in range(_NBUF):
            pltpu.make_async_copy(
                obuf[b], out_dst(n_win - _NBUF + b), osem[b]
            ).wait()

    out = _gather(tab2, xt)
    return jnp.transpose(out, (2, 4, 0, 1, 3)).reshape(B, S, E)


# speedup vs baseline: 1.2917x; 1.2917x over previous
import jax
import jax.numpy as jnp
from jax import lax
from jax.experimental import pallas as pl
from jax.experimental.pallas import tpu as pltpu
from jax.experimental.pallas import tpu_sc as plsc

_EMB = 64
_SCALE = 8.0
_W = 128
_NBUF = 4
_NC, _NS = 2, 16
_NWORK = _NC * _NS
_LANES = 16


def kernel(x, table):
    B, S = x.shape
    V, E = table.shape
    n = B * S
    n_win = n // (_W * _NWORK)
    tab2 = jnp.pad(table, ((0, 0), (0, E)))
    idx = x.reshape(n // _W, _W)
    mesh = plsc.VectorSubcoreMesh(core_axis_name="c", subcore_axis_name="s")

    @pl.kernel(
        out_type=jax.ShapeDtypeStruct((n, E), table.dtype),
        mesh=mesh,
        compiler_params=pltpu.CompilerParams(use_tc_tiling_on_sc=False),
        scratch_types=(
            [pltpu.VMEM((n_win, _W), jnp.int32)]
            + [pltpu.VMEM((_W, 2 * E), jnp.float32) for _ in range(_NBUF)]
            + [pltpu.VMEM((_W, E), jnp.float32) for _ in range(_NBUF)]
            + [pltpu.SemaphoreType.DMA for _ in range(2 * _NBUF + 1)]
        ),
    )
    def _gather(tab_hbm, i_hbm, o_hbm, idx_v, *scr):
        gbuf = scr[:_NBUF]
        obuf = scr[_NBUF : 2 * _NBUF]
        gsem = scr[2 * _NBUF : 3 * _NBUF]
        osem = scr[3 * _NBUF : 4 * _NBUF]
        isem = scr[4 * _NBUF]

        wid = lax.axis_index("s") * _NC + lax.axis_index("c")
        win0 = wid * n_win

        pltpu.async_copy(i_hbm.at[pl.ds(win0, n_win)], idx_v, isem).wait()

        def start_gather(b, g):
            pltpu.make_async_copy(
                tab_hbm.at[idx_v.at[g]], gbuf[b], gsem[b]
            ).start()

        for b in range(_NBUF):
            start_gather(b, b)

        @pl.loop(0, n_win, step=_NBUF)
        def _round(t):
            for b in range(_NBUF):
                g = t + b
                pltpu.make_async_copy(
                    tab_hbm.at[idx_v.at[g]], gbuf[b], gsem[b]
                ).wait()

                @pl.when(t > 0)
                def _():
                    pltpu.make_async_copy(
                        obuf[b], o_hbm.at[pl.ds((win0 + g) * _W, _W)], osem[b]
                    ).wait()

                @pl.loop(0, _W)
                def _row(r):
                    for c in range(0, E, _LANES):
                        obuf[b].at[r, pl.ds(c, _LANES)][...] = (
                            gbuf[b].at[r, pl.ds(c, _LANES)][...] * _SCALE
                        )

                @pl.when(g + _NBUF < n_win)
                def _():
                    start_gather(b, g + _NBUF)

                pltpu.make_async_copy(
                    obuf[b], o_hbm.at[pl.ds((win0 + g) * _W, _W)], osem[b]
                ).start()

        for b in range(_NBUF):
            pltpu.make_async_copy(
                obuf[b],
                o_hbm.at[pl.ds((win0 + n_win - _NBUF + b) * _W, _W)],
                osem[b],
            ).wait()

    out = _gather(tab2, idx)
    return out.reshape(B, S, E)


# speedup vs baseline: 1.5830x; 1.2256x over previous
import jax
import jax.numpy as jnp
from jax import lax
from jax.experimental import pallas as pl
from jax.experimental.pallas import tpu as pltpu
from jax.experimental.pallas import tpu_sc as plsc

_EMB = 64
_SCALE = 8.0
_W = 128
_NBUF = 4
_NC, _NS = 2, 16
_NWORK = _NC * _NS
_LANES = 16


def kernel(x, table):
    B, S = x.shape
    V, E = table.shape
    n = B * S
    n_win = n // (_W * _NWORK)
    tab2 = table
    idx = x.reshape(n // _W, _W)
    mesh = plsc.VectorSubcoreMesh(core_axis_name="c", subcore_axis_name="s")

    @pl.kernel(
        out_type=jax.ShapeDtypeStruct((n, E), table.dtype),
        mesh=mesh,
        compiler_params=pltpu.CompilerParams(use_tc_tiling_on_sc=False),
        scratch_types=(
            [pltpu.VMEM((n_win, _W), jnp.int32)]
            + [pltpu.VMEM((_W, E), jnp.float32) for _ in range(_NBUF)]
            + [pltpu.VMEM((_W, E), jnp.float32) for _ in range(_NBUF)]
            + [pltpu.SemaphoreType.DMA for _ in range(2 * _NBUF + 1)]
        ),
    )
    def _gather(tab_hbm, i_hbm, o_hbm, idx_v, *scr):
        gbuf = scr[:_NBUF]
        obuf = scr[_NBUF : 2 * _NBUF]
        gsem = scr[2 * _NBUF : 3 * _NBUF]
        osem = scr[3 * _NBUF : 4 * _NBUF]
        isem = scr[4 * _NBUF]

        wid = lax.axis_index("s") * _NC + lax.axis_index("c")
        win0 = wid * n_win

        pltpu.async_copy(i_hbm.at[pl.ds(win0, n_win)], idx_v, isem).wait()

        def start_gather(b, g):
            pltpu.make_async_copy(
                tab_hbm.at[idx_v.at[g]], gbuf[b], gsem[b]
            ).start()

        for b in range(_NBUF):
            start_gather(b, b)

        @pl.loop(0, n_win, step=_NBUF)
        def _round(t):
            for b in range(_NBUF):
                g = t + b
                pltpu.make_async_copy(
                    tab_hbm.at[idx_v.at[g]], gbuf[b], gsem[b]
                ).wait()

                @pl.when(t > 0)
                def _():
                    pltpu.make_async_copy(
                        obuf[b], o_hbm.at[pl.ds((win0 + g) * _W, _W)], osem[b]
                    ).wait()

                @pl.loop(0, _W)
                def _row(r):
                    for c in range(0, E, _LANES):
                        obuf[b].at[r, pl.ds(c, _LANES)][...] = (
                            gbuf[b].at[r, pl.ds(c, _LANES)][...] * _SCALE
                        )

                @pl.when(g + _NBUF < n_win)
                def _():
                    start_gather(b, g + _NBUF)

                pltpu.make_async_copy(
                    obuf[b], o_hbm.at[pl.ds((win0 + g) * _W, _W)], osem[b]
                ).start()

        for b in range(_NBUF):
            pltpu.make_async_copy(
                obuf[b],
                o_hbm.at[pl.ds((win0 + n_win - _NBUF + b) * _W, _W)],
                osem[b],
            ).wait()

    out = _gather(tab2, idx)
    return out.reshape(B, S, E)
